# baseline (speedup 1.0000x reference)
import functools

import jax
import jax.numpy as jnp
from jax import lax
from jax.experimental import pallas as pl
from jax.experimental.pallas import tpu as pltpu
from jax.experimental.pallas import tpu_sc as plsc

_N_CORES = 2
_N_SUBCORES = 16
_LANES = 16
_NW = _N_CORES * _N_SUBCORES
_K = 128
_NCH = 78
_G = 26


def _sc_gather_scale_scatter(x, src, dst, w, src_t, dst_t, w_t):
    n, d = x.shape
    nvec = d // _LANES
    rowch = 16
    mesh = plsc.VectorSubcoreMesh(core_axis_name="c", subcore_axis_name="s",
                                  num_cores=_N_CORES, num_subcores=_N_SUBCORES)

    @functools.partial(
        pl.kernel,
        out_type=jax.ShapeDtypeStruct((_N_CORES, n, d), jnp.float32),
        mesh=mesh,
        scratch_types=[
            pltpu.VMEM((_G, _K), jnp.int32),
            pltpu.VMEM((_G, _K), jnp.int32),
            pltpu.VMEM((_G, _K), jnp.float32),
            pltpu.VMEM((_K, d), jnp.float32),
            pltpu.VMEM((_K, d), jnp.float32),
            pltpu.VMEM_SHARED((n, d), jnp.float32),
            pltpu.SemaphoreType.DMA,
            pltpu.SemaphoreType.DMA,
        ],
    )
    def sc_kernel(x_hbm, src_hbm, dst_hbm, w_hbm, st_hbm, dt_hbm, wt_hbm,
                  out_hbm, src_v, dst_v, w_v, buf_a, buf_b, acc,
                  sem_a, sem_b):
        cid = lax.axis_index("c")
        sid = lax.axis_index("s")
        wid = sid * _N_CORES + cid

        @pl.loop(0, _K)
        def _zero_rows(r):
            for v in range(nvec):
                buf_a[r, pl.ds(v * _LANES, _LANES)] = jnp.zeros(
                    (_LANES,), jnp.float32)

        base = pl.multiple_of(sid * 624, 8)
        for q in range(4):
            pltpu.sync_copy(buf_a, acc.at[pl.ds(base + q * _K, _K)])

        @pl.when(sid < _N_SUBCORES - 1)
        def _():
            pltpu.sync_copy(buf_a.at[pl.ds(0, 112)],
                            acc.at[pl.ds(base + 512, 112)])

        @pl.when(sid == _N_SUBCORES - 1)
        def _():
            pltpu.sync_copy(buf_a, acc.at[pl.ds(base + 512, _K)])

        plsc.subcore_barrier()

        def gather_start(c, buf, sem):
            pltpu.async_copy(x_hbm.at[src_v.at[c]], buf, sem)

        def gather_wait(c, buf, sem):
            pltpu.make_async_copy(x_hbm.at[src_v.at[c]], buf, sem).wait()

        def scale(c, buf):
            @pl.loop(0, _K, step=_LANES)
            def _scale(g0):
                g = pl.multiple_of(g0, _LANES)
                w16 = w_v[c, pl.ds(g, _LANES)]
                for j in range(_LANES):
                    ws = w16[j]
                    for v in range(nvec):
                        sl = (g + j, pl.ds(v * _LANES, _LANES))
                        buf[sl] = buf[sl] * ws

        def scale_scatter(c, buf):
            scale(c, buf)
            pltpu.sync_copy(buf, acc.at[dst_v.at[c]], add=True)

        @pl.loop(0, _NCH // _G)
        def _blocks(b):
            pltpu.sync_copy(src_hbm.at[wid, b], src_v)
            pltpu.sync_copy(dst_hbm.at[wid, b], dst_v)
            pltpu.sync_copy(w_hbm.at[wid, b], w_v)

            gather_start(0, buf_a, sem_a)

            @pl.loop(0, _G - 2, step=2)
            def _edges(c):
                gather_wait(c, buf_a, sem_a)
                gather_start(c + 1, buf_b, sem_b)
                scale_scatter(c, buf_a)
                gather_wait(c + 1, buf_b, sem_b)
                gather_start(c + 2, buf_a, sem_a)
                scale_scatter(c + 1, buf_b)

            gather_wait(_G - 2, buf_a, sem_a)
            gather_start(_G - 1, buf_b, sem_b)
            scale_scatter(_G - 2, buf_a)
            gather_wait(_G - 1, buf_b, sem_b)
            scale_scatter(_G - 1, buf_b)

        pltpu.sync_copy(st_hbm.at[wid], src_v.at[pl.ds(0, 1)])
        pltpu.sync_copy(dt_hbm.at[wid], dst_v.at[pl.ds(0, 1)])
        pltpu.sync_copy(wt_hbm.at[wid], w_v.at[pl.ds(0, 1)])
        gather_start(0, buf_a, sem_a)
        gather_wait(0, buf_a, sem_a)
        scale_scatter(0, buf_a)

        plsc.subcore_barrier()

        for q in range(4):
            pltpu.sync_copy(acc.at[pl.ds(base + q * _K, _K)],
                            out_hbm.at[cid, pl.ds(base + q * _K, _K)])

        @pl.when(sid < _N_SUBCORES - 1)
        def _():
            pltpu.sync_copy(acc.at[pl.ds(base + 512, 112)],
                            out_hbm.at[cid, pl.ds(base + 512, 112)])

        @pl.when(sid == _N_SUBCORES - 1)
        def _():
            pltpu.sync_copy(acc.at[pl.ds(base + 512, _K)],
                            out_hbm.at[cid, pl.ds(base + 512, _K)])

    return sc_kernel(x, src, dst, w, src_t, dst_t, w_t)


def _combine_partials(partials):
    _, n, d = partials.shape
    blk = 2000

    def body(p_ref, o_ref):
        o_ref[...] = p_ref[0] + p_ref[1]

    return pl.pallas_call(
        body,
        out_shape=jax.ShapeDtypeStruct((n, d), jnp.float32),
        grid=(n // blk,),
        in_specs=[pl.BlockSpec((2, blk, d), lambda i: (0, i, 0))],
        out_specs=pl.BlockSpec((blk, d), lambda i: (i, 0)),
    )(partials)


def kernel(x, edge_index, w, mode):
    del mode
    n = x.shape[0]
    e = edge_index.shape[1]
    e_main = _NW * _NCH * _K
    pad = _NW * _K - (e - e_main)
    nblk = _NCH // _G
    src = edge_index[0]
    dst = edge_index[1]
    wf = w[:, 0]
    src_m = src[:e_main].reshape(_NW, nblk, _G, _K)
    dst_m = dst[:e_main].reshape(_NW, nblk, _G, _K)
    w_m = wf[:e_main].reshape(_NW, nblk, _G, _K)
    fill = jnp.arange(pad, dtype=jnp.int32) % n
    src_t = jnp.concatenate([src[e_main:], fill]).reshape(_NW, 1, _K)
    dst_t = jnp.concatenate([dst[e_main:], fill]).reshape(_NW, 1, _K)
    w_t = jnp.concatenate(
        [wf[e_main:], jnp.zeros((pad,), jnp.float32)]).reshape(_NW, 1, _K)
    partials = _sc_gather_scale_scatter(x, src_m, dst_m, w_m,
                                        src_t, dst_t, w_t)
    return _combine_partials(partials)


# speedup vs baseline: 1.1807x; 1.1807x over previous
import functools

import jax
import jax.numpy as jnp
from jax import lax
from jax.experimental import pallas as pl
from jax.experimental.pallas import tpu as pltpu
from jax.experimental.pallas import tpu_sc as plsc

_N_CORES = 2
_N_SUBCORES = 16
_LANES = 16
_NW = _N_CORES * _N_SUBCORES
_K = 128
_NCH = 80
_G = 40


def _sc_gather_scale_scatter(x, src, dst, w):
    n, d = x.shape
    nvec = d // _LANES
    rowch = 16
    mesh = plsc.VectorSubcoreMesh(core_axis_name="c", subcore_axis_name="s",
                                  num_cores=_N_CORES, num_subcores=_N_SUBCORES)

    @functools.partial(
        pl.kernel,
        out_type=jax.ShapeDtypeStruct((_N_CORES, n, d), jnp.float32),
        mesh=mesh,
        scratch_types=[
            pltpu.VMEM((_G, _K), jnp.int32),
            pltpu.VMEM((_G, _K), jnp.int32),
            pltpu.VMEM((_G, _K), jnp.float32),
            pltpu.VMEM((_K, d), jnp.float32),
            pltpu.VMEM((_K, d), jnp.float32),
            pltpu.VMEM_SHARED((n, d), jnp.float32),
            pltpu.SemaphoreType.DMA,
            pltpu.SemaphoreType.DMA,
        ],
    )
    def sc_kernel(x_hbm, src_hbm, dst_hbm, w_hbm, out_hbm,
                  src_v, dst_v, w_v, buf_a, buf_b, acc, sem_a, sem_b):
        cid = lax.axis_index("c")
        sid = lax.axis_index("s")
        wid = sid * _N_CORES + cid

        @pl.loop(0, _K)
        def _zero_rows(r):
            for v in range(nvec):
                buf_a[r, pl.ds(v * _LANES, _LANES)] = jnp.zeros(
                    (_LANES,), jnp.float32)

        base = pl.multiple_of(sid * 624, 8)
        for q in range(4):
            pltpu.sync_copy(buf_a, acc.at[pl.ds(base + q * _K, _K)])

        @pl.when(sid < _N_SUBCORES - 1)
        def _():
            pltpu.sync_copy(buf_a.at[pl.ds(0, 112)],
                            acc.at[pl.ds(base + 512, 112)])

        @pl.when(sid == _N_SUBCORES - 1)
        def _():
            pltpu.sync_copy(buf_a, acc.at[pl.ds(base + 512, _K)])

        plsc.subcore_barrier()

        def gather_start(c, buf, sem):
            pltpu.async_copy(x_hbm.at[src_v.at[c]], buf, sem)

        def gather_wait(c, buf, sem):
            pltpu.make_async_copy(x_hbm.at[src_v.at[c]], buf, sem).wait()

        def scale(c, buf):
            @pl.loop(0, _K, step=_LANES)
            def _scale(g0):
                g = pl.multiple_of(g0, _LANES)
                w16 = w_v[c, pl.ds(g, _LANES)]
                for j in range(_LANES):
                    ws = w16[j]
                    for v in range(nvec):
                        sl = (g + j, pl.ds(v * _LANES, _LANES))
                        buf[sl] = buf[sl] * ws

        def scale_scatter(c, buf):
            scale(c, buf)
            pltpu.sync_copy(buf, acc.at[dst_v.at[c]], add=True)

        @pl.loop(0, _NCH // _G)
        def _blocks(b):
            pltpu.sync_copy(src_hbm.at[wid, b], src_v)
            pltpu.sync_copy(dst_hbm.at[wid, b], dst_v)
            pltpu.sync_copy(w_hbm.at[wid, b], w_v)

            gather_start(0, buf_a, sem_a)

            @pl.loop(0, _G - 2, step=2)
            def _edges(c):
                gather_wait(c, buf_a, sem_a)
                gather_start(c + 1, buf_b, sem_b)
                scale_scatter(c, buf_a)
                gather_wait(c + 1, buf_b, sem_b)
                gather_start(c + 2, buf_a, sem_a)
                scale_scatter(c + 1, buf_b)

            gather_wait(_G - 2, buf_a, sem_a)
            gather_start(_G - 1, buf_b, sem_b)
            scale_scatter(_G - 2, buf_a)
            gather_wait(_G - 1, buf_b, sem_b)
            scale_scatter(_G - 1, buf_b)

        plsc.subcore_barrier()

        for q in range(4):
            pltpu.sync_copy(acc.at[pl.ds(base + q * _K, _K)],
                            out_hbm.at[cid, pl.ds(base + q * _K, _K)])

        @pl.when(sid < _N_SUBCORES - 1)
        def _():
            pltpu.sync_copy(acc.at[pl.ds(base + 512, 112)],
                            out_hbm.at[cid, pl.ds(base + 512, 112)])

        @pl.when(sid == _N_SUBCORES - 1)
        def _():
            pltpu.sync_copy(acc.at[pl.ds(base + 512, _K)],
                            out_hbm.at[cid, pl.ds(base + 512, _K)])

    return sc_kernel(x, src, dst, w)


def _combine_partials(partials):
    _, n, d = partials.shape
    blk = 2000

    def body(p_ref, o_ref):
        o_ref[...] = p_ref[0] + p_ref[1]

    return pl.pallas_call(
        body,
        out_shape=jax.ShapeDtypeStruct((n, d), jnp.float32),
        grid=(n // blk,),
        in_specs=[pl.BlockSpec((2, blk, d), lambda i: (0, i, 0))],
        out_specs=pl.BlockSpec((blk, d), lambda i: (i, 0)),
    )(partials)


def kernel(x, edge_index, w, mode):
    del mode
    n = x.shape[0]
    e = edge_index.shape[1]
    e_pad = _NW * _NCH * _K
    pad = e_pad - e
    fill = (jnp.arange(pad, dtype=jnp.int32) % n) if pad else None
    src = edge_index[0]
    dst = edge_index[1]
    wf = w[:, 0]
    if pad:
        src = jnp.concatenate([src, fill])
        dst = jnp.concatenate([dst, fill])
        wf = jnp.concatenate([wf, jnp.zeros((pad,), jnp.float32)])
    nblk = _NCH // _G
    src = src.reshape(_NW, nblk, _G, _K)
    dst = dst.reshape(_NW, nblk, _G, _K)
    wf = wf.reshape(_NW, nblk, _G, _K)
    partials = _sc_gather_scale_scatter(x, src, dst, wf)
    return _combine_partials(partials)


# speedup vs baseline: 1.1969x; 1.0138x over previous
import functools

import jax
import jax.numpy as jnp
from jax import lax
from jax.experimental import pallas as pl
from jax.experimental.pallas import tpu as pltpu
from jax.experimental.pallas import tpu_sc as plsc

_N_CORES = 2
_N_SUBCORES = 16
_LANES = 16
_NW = _N_CORES * _N_SUBCORES
_K = 128
_NCH = 80
_G = 40


def _sc_gather_scale_scatter(x, src, dst, w):
    n, d = x.shape
    nvec = d // _LANES
    rowch = 16
    mesh = plsc.VectorSubcoreMesh(core_axis_name="c", subcore_axis_name="s",
                                  num_cores=_N_CORES, num_subcores=_N_SUBCORES)

    @functools.partial(
        pl.kernel,
        out_type=jax.ShapeDtypeStruct((_N_CORES, n, d), jnp.float32),
        mesh=mesh,
        scratch_types=[
            pltpu.VMEM((_G, _K), jnp.int32),
            pltpu.VMEM((_G, _K), jnp.int32),
            pltpu.VMEM((_G, _K), jnp.float32),
            pltpu.VMEM((_K, d), jnp.float32),
            pltpu.VMEM((_K, d), jnp.float32),
            pltpu.VMEM_SHARED((n, d), jnp.float32),
            pltpu.SemaphoreType.DMA,
            pltpu.SemaphoreType.DMA,
            pltpu.SemaphoreType.DMA,
        ],
    )
    def sc_kernel(x_hbm, src_hbm, dst_hbm, w_hbm, out_hbm,
                  src_v, dst_v, w_v, buf_a, buf_b, acc, sem_a, sem_b,
                  sem_i):
        cid = lax.axis_index("c")
        sid = lax.axis_index("s")
        wid = sid * _N_CORES + cid

        @pl.loop(0, _K)
        def _zero_rows(r):
            for v in range(nvec):
                buf_a[r, pl.ds(v * _LANES, _LANES)] = jnp.zeros(
                    (_LANES,), jnp.float32)

        base = pl.multiple_of(sid * 624, 8)
        for q in range(4):
            pltpu.sync_copy(buf_a, acc.at[pl.ds(base + q * _K, _K)])

        @pl.when(sid < _N_SUBCORES - 1)
        def _():
            pltpu.sync_copy(buf_a.at[pl.ds(0, 112)],
                            acc.at[pl.ds(base + 512, 112)])

        @pl.when(sid == _N_SUBCORES - 1)
        def _():
            pltpu.sync_copy(buf_a, acc.at[pl.ds(base + 512, _K)])

        plsc.subcore_barrier()

        def gather_start(c, buf, sem):
            pltpu.async_copy(x_hbm.at[src_v.at[c]], buf, sem)

        def gather_wait(c, buf, sem):
            pltpu.make_async_copy(x_hbm.at[src_v.at[c]], buf, sem).wait()

        def scale(c, buf):
            @pl.loop(0, _K, step=_LANES)
            def _scale(g0):
                g = pl.multiple_of(g0, _LANES)
                w16 = w_v[c, pl.ds(g, _LANES)]
                for j in range(_LANES):
                    ws = w16[j]
                    for v in range(nvec):
                        sl = (g + j, pl.ds(v * _LANES, _LANES))
                        buf[sl] = buf[sl] * ws

        def scale_scatter(c, buf):
            scale(c, buf)
            pltpu.sync_copy(buf, acc.at[dst_v.at[c]], add=True)

        @pl.loop(0, _NCH // _G)
        def _blocks(b):
            pltpu.async_copy(src_hbm.at[wid, b], src_v, sem_i)
            pltpu.async_copy(dst_hbm.at[wid, b], dst_v, sem_i)
            pltpu.async_copy(w_hbm.at[wid, b], w_v, sem_i)
            pltpu.make_async_copy(src_hbm.at[wid, b], src_v, sem_i).wait()
            pltpu.make_async_copy(dst_hbm.at[wid, b], dst_v, sem_i).wait()
            pltpu.make_async_copy(w_hbm.at[wid, b], w_v, sem_i).wait()

            gather_start(0, buf_a, sem_a)

            @pl.loop(0, _G - 2, step=2)
            def _edges(c):
                gather_wait(c, buf_a, sem_a)
                gather_start(c + 1, buf_b, sem_b)
                scale_scatter(c, buf_a)
                gather_wait(c + 1, buf_b, sem_b)
                gather_start(c + 2, buf_a, sem_a)
                scale_scatter(c + 1, buf_b)

            gather_wait(_G - 2, buf_a, sem_a)
            gather_start(_G - 1, buf_b, sem_b)
            scale_scatter(_G - 2, buf_a)
            gather_wait(_G - 1, buf_b, sem_b)
            scale_scatter(_G - 1, buf_b)

        plsc.subcore_barrier()

        for q in range(4):
            pltpu.sync_copy(acc.at[pl.ds(base + q * _K, _K)],
                            out_hbm.at[cid, pl.ds(base + q * _K, _K)])

        @pl.when(sid < _N_SUBCORES - 1)
        def _():
            pltpu.sync_copy(acc.at[pl.ds(base + 512, 112)],
                            out_hbm.at[cid, pl.ds(base + 512, 112)])

        @pl.when(sid == _N_SUBCORES - 1)
        def _():
            pltpu.sync_copy(acc.at[pl.ds(base + 512, _K)],
                            out_hbm.at[cid, pl.ds(base + 512, _K)])

    return sc_kernel(x, src, dst, w)


def _combine_partials(partials):
    _, n, d = partials.shape
    blk = 2000

    def body(p_ref, o_ref):
        o_ref[...] = p_ref[0] + p_ref[1]

    return pl.pallas_call(
        body,
        out_shape=jax.ShapeDtypeStruct((n, d), jnp.float32),
        grid=(n // blk,),
        in_specs=[pl.BlockSpec((2, blk, d), lambda i: (0, i, 0))],
        out_specs=pl.BlockSpec((blk, d), lambda i: (i, 0)),
    )(partials)


def kernel(x, edge_index, w, mode):
    del mode
    n = x.shape[0]
    e = edge_index.shape[1]
    e_pad = _NW * _NCH * _K
    pad = e_pad - e
    fill = (jnp.arange(pad, dtype=jnp.int32) % n) if pad else None
    src = edge_index[0]
    dst = edge_index[1]
    wf = w[:, 0]
    if pad:
        src = jnp.concatenate([src, fill])
        dst = jnp.concatenate([dst, fill])
        wf = jnp.concatenate([wf, jnp.zeros((pad,), jnp.float32)])
    nblk = _NCH // _G
    src = src.reshape(_NW, nblk, _G, _K)
    dst = dst.reshape(_NW, nblk, _G, _K)
    wf = wf.reshape(_NW, nblk, _G, _K)
    partials = _sc_gather_scale_scatter(x, src, dst, wf)
    return _combine_partials(partials)
